# baseline (speedup 1.0000x reference)
import functools

import jax
import jax.numpy as jnp
from jax import lax
from jax.experimental import pallas as pl
from jax.experimental.pallas import tpu as pltpu
from jax.experimental.pallas import tpu_sc as plsc

MAX_RATING = 5.0
MIN_RATING = 1.0

B = 16384
D = 128
NH = 128

_NC = 2
_NS = 16
_NW = _NC * _NS
_BPW = B // _NW
_CHUNK = 128
_NCHUNK = _BPW // _CHUNK


_HALF = _BPW // 2
_HCHUNK = _HALF // _CHUNK


@functools.cache
def _make_gather():
    mesh = plsc.VectorSubcoreMesh(core_axis_name="c", subcore_axis_name="s")

    @functools.partial(
        pl.kernel,
        mesh=mesh,
        out_type=[
            jax.ShapeDtypeStruct((B, D), jnp.float32),
            jax.ShapeDtypeStruct((B, D), jnp.float32),
        ],
        scratch_types=[
            pltpu.VMEM((_HCHUNK, _CHUNK), jnp.int32),
            pltpu.VMEM((_HALF, D), jnp.float32),
            pltpu.VMEM((_HALF, D), jnp.float32),
            pltpu.SemaphoreType.DMA,
            pltpu.SemaphoreType.DMA,
        ],
    )
    def gather2(u_tab, m_tab, users, movies, u_out, m_out,
                idx_v, rows0, rows1, gsem, wsem):
        wid = lax.axis_index("s") * _NC + lax.axis_index("c")
        base = wid * _BPW
        bufs = (rows0, rows1)
        phases = [(u_tab, users, u_out, 0), (u_tab, users, u_out, _HALF),
                  (m_tab, movies, m_out, 0), (m_tab, movies, m_out, _HALF)]
        writes = [None, None]
        for p, (tab, idx_hbm, out_hbm, off) in enumerate(phases):
            buf = bufs[p % 2]
            if writes[p % 2] is not None:
                writes[p % 2].wait()
            for j in range(_HCHUNK):
                pltpu.sync_copy(
                    idx_hbm.at[pl.ds(base + off + j * _CHUNK, _CHUNK)],
                    idx_v.at[j])
            gathers = [
                pltpu.async_copy(tab.at[idx_v.at[j]],
                                 buf.at[pl.ds(j * _CHUNK, _CHUNK)], gsem)
                for j in range(_HCHUNK)
            ]
            for g in gathers:
                g.wait()
            writes[p % 2] = pltpu.async_copy(
                buf, out_hbm.at[pl.ds(base + off, _HALF)], wsem)
        for w in writes:
            w.wait()

    return gather2


_TILE = 2048


def _mlp_body(u2_ref, m2_ref, w1u_ref, w1m_ref, b1_ref, w2_ref, b2_ref, out_ref):
    h = (jnp.dot(u2_ref[...], w1u_ref[...], preferred_element_type=jnp.float32)
         + jnp.dot(m2_ref[...], w1m_ref[...], preferred_element_type=jnp.float32)
         + b1_ref[...])
    h = jnp.maximum(h, 0.0)
    z = jnp.sum(h * w2_ref[...], axis=1, keepdims=True) + b2_ref[...]
    out_ref[...] = (jax.nn.sigmoid(z) * (MAX_RATING - MIN_RATING) + MIN_RATING)


def _mlp(u2, m2, w1u, w1m, b1, w2, b2):
    grid = (B // _TILE,)
    return pl.pallas_call(
        _mlp_body,
        grid=grid,
        in_specs=[
            pl.BlockSpec((_TILE, D), lambda i: (i, 0)),
            pl.BlockSpec((_TILE, D), lambda i: (i, 0)),
            pl.BlockSpec((D, NH), lambda i: (0, 0)),
            pl.BlockSpec((D, NH), lambda i: (0, 0)),
            pl.BlockSpec((1, NH), lambda i: (0, 0)),
            pl.BlockSpec((1, NH), lambda i: (0, 0)),
            pl.BlockSpec((1, 1), lambda i: (0, 0)),
        ],
        out_specs=pl.BlockSpec((_TILE, 1), lambda i: (i, 0)),
        out_shape=jax.ShapeDtypeStruct((B, 1), jnp.float32),
    )(u2, m2, w1u, w1m, b1, w2, b2)


def kernel(users, movies, u_weight, m_weight, lin1_w, lin1_b, lin2_w, lin2_b):
    u2, m2 = _make_gather()(u_weight, m_weight, users, movies)
    w1u = lin1_w[:, :D].T
    w1m = lin1_w[:, D:].T
    b1 = lin1_b.reshape(1, NH)
    w2 = lin2_w.reshape(1, NH)
    b2 = lin2_b.reshape(1, 1)
    return _mlp(u2, m2, w1u, w1m, b1, w2, b2)


# speedup vs baseline: 1.1732x; 1.1732x over previous
import functools

import jax
import jax.numpy as jnp
from jax import lax
from jax.experimental import pallas as pl
from jax.experimental.pallas import tpu as pltpu
from jax.experimental.pallas import tpu_sc as plsc

MAX_RATING = 5.0
MIN_RATING = 1.0

B = 16384
D = 128
NH = 128

_NC = 2
_NS = 16
_NW = _NC * _NS
_BPW = B // _NW
_CHUNK = 128
_NCHUNK = _BPW // _CHUNK


_HALF = _BPW // 2
_HCHUNK = _HALF // _CHUNK


@functools.cache
def _make_gather():
    mesh = plsc.VectorSubcoreMesh(core_axis_name="c", subcore_axis_name="s")

    @functools.partial(
        pl.kernel,
        mesh=mesh,
        out_type=[
            jax.ShapeDtypeStruct((B, D), jnp.float32),
            jax.ShapeDtypeStruct((B, D), jnp.float32),
        ],
        scratch_types=[
            pltpu.VMEM((_HCHUNK, _CHUNK), jnp.int32),
            pltpu.VMEM((_HALF, D), jnp.float32),
            pltpu.VMEM((_HALF, D), jnp.float32),
            pltpu.SemaphoreType.DMA,
            pltpu.SemaphoreType.DMA,
        ],
    )
    def gather2(u_tab, m_tab, users, movies, u_out, m_out,
                idx_v, rows0, rows1, gsem, wsem):
        wid = lax.axis_index("s") * _NC + lax.axis_index("c")
        base = wid * _BPW
        bufs = (rows0, rows1)
        phases = [(u_tab, users, u_out, 0), (u_tab, users, u_out, _HALF),
                  (m_tab, movies, m_out, 0), (m_tab, movies, m_out, _HALF)]
        writes = [None, None]
        for p, (tab, idx_hbm, out_hbm, off) in enumerate(phases):
            buf = bufs[p % 2]
            if writes[p % 2] is not None:
                writes[p % 2].wait()
            for j in range(_HCHUNK):
                pltpu.sync_copy(
                    idx_hbm.at[pl.ds(base + off + j * _CHUNK, _CHUNK)],
                    idx_v.at[j])
            gathers = [
                pltpu.async_copy(tab.at[idx_v.at[j]],
                                 buf.at[pl.ds(j * _CHUNK, _CHUNK)], gsem)
                for j in range(_HCHUNK)
            ]
            for g in gathers:
                g.wait()
            writes[p % 2] = pltpu.async_copy(
                buf, out_hbm.at[pl.ds(base + off, _HALF)], wsem)
        for w in writes:
            w.wait()

    return gather2


_TILE = 2048


def _mlp_body(u2_ref, m2_ref, w1u_ref, w1m_ref, b1_ref, w2_ref, b2_ref, out_ref):
    h = (jnp.dot(u2_ref[...], w1u_ref[...], preferred_element_type=jnp.float32)
         + jnp.dot(m2_ref[...], w1m_ref[...], preferred_element_type=jnp.float32)
         + b1_ref[...])
    h = jnp.maximum(h, 0.0)
    z = jnp.sum(h * w2_ref[...], axis=1) + b2_ref[0, 0]
    r = jax.nn.sigmoid(z) * (MAX_RATING - MIN_RATING) + MIN_RATING
    out_ref[...] = r.reshape(_TILE // 128, 128)


def _mlp(u2, m2, w1u, w1m, b1, w2, b2):
    grid = (B // _TILE,)
    return pl.pallas_call(
        _mlp_body,
        grid=grid,
        in_specs=[
            pl.BlockSpec((_TILE, D), lambda i: (i, 0)),
            pl.BlockSpec((_TILE, D), lambda i: (i, 0)),
            pl.BlockSpec((D, NH), lambda i: (0, 0)),
            pl.BlockSpec((D, NH), lambda i: (0, 0)),
            pl.BlockSpec((1, NH), lambda i: (0, 0)),
            pl.BlockSpec((1, NH), lambda i: (0, 0)),
            pl.BlockSpec((1, 1), lambda i: (0, 0)),
        ],
        out_specs=pl.BlockSpec((_TILE // 128, 128), lambda i: (i, 0)),
        out_shape=jax.ShapeDtypeStruct((B // 128, 128), jnp.float32),
    )(u2, m2, w1u, w1m, b1, w2, b2)


def kernel(users, movies, u_weight, m_weight, lin1_w, lin1_b, lin2_w, lin2_b):
    u2, m2 = _make_gather()(u_weight, m_weight, users, movies)
    w1u = lin1_w[:, :D].T
    w1m = lin1_w[:, D:].T
    b1 = lin1_b.reshape(1, NH)
    w2 = lin2_w.reshape(1, NH)
    b2 = lin2_b.reshape(1, 1)
    return _mlp(u2, m2, w1u, w1m, b1, w2, b2).reshape(B, 1)


# speedup vs baseline: 1.3444x; 1.1459x over previous
import functools

import jax
import jax.numpy as jnp
from jax import lax
from jax.experimental import pallas as pl
from jax.experimental.pallas import tpu as pltpu
from jax.experimental.pallas import tpu_sc as plsc

MAX_RATING = 5.0
MIN_RATING = 1.0

B = 16384
D = 128
NH = 128

_NC = 2
_NS = 16
_NW = _NC * _NS
_BPW = B // _NW
_CHUNK = 128
_NCHUNK = _BPW // _CHUNK


_NGATH = 2 * _NCHUNK
_NBUF = 7
_IPW = B // 128 // _NW


@functools.cache
def _make_gather():
    mesh = plsc.VectorSubcoreMesh(core_axis_name="c", subcore_axis_name="s")

    @functools.partial(
        pl.kernel,
        mesh=mesh,
        out_type=[
            jax.ShapeDtypeStruct((B, D), jnp.float32),
            jax.ShapeDtypeStruct((B, D), jnp.float32),
        ],
        scratch_types=(
            [pltpu.VMEM((_IPW, _CHUNK), jnp.int32)] * 2
            + [pltpu.VMEM((_CHUNK, D), jnp.float32)] * _NBUF
            + [pltpu.SemaphoreType.DMA] * (2 * _NGATH)
        ),
    )
    def gather2(u_tab, m_tab, users2d, movies2d, u_out, m_out, *scratch):
        iu, im = scratch[:2]
        bufs = scratch[2:2 + _NBUF]
        gsems = scratch[2 + _NBUF:2 + _NBUF + _NGATH]
        wsems = scratch[2 + _NBUF + _NGATH:]
        wid = lax.axis_index("s") * _NC + lax.axis_index("c")
        base = wid * _BPW
        pltpu.sync_copy(users2d.at[pl.ds(wid * _IPW, _IPW)], iu)
        pltpu.sync_copy(movies2d.at[pl.ds(wid * _IPW, _IPW)], im)
        specs = [(u_tab, iu, u_out, j) for j in range(_NCHUNK)] + \
                [(m_tab, im, m_out, j) for j in range(_NCHUNK)]
        gathers = [None] * _NGATH
        writes = [None] * _NGATH
        for j in range(min(_NBUF, _NGATH)):
            tab, idx, _, c = specs[j]
            gathers[j] = pltpu.async_copy(tab.at[idx.at[c]], bufs[j], gsems[j])
        for j in range(_NGATH):
            if j >= _NBUF:
                writes[j - _NBUF].wait()
                tab, idx, _, c = specs[j]
                gathers[j] = pltpu.async_copy(tab.at[idx.at[c]],
                                              bufs[j % _NBUF], gsems[j])
            gathers[j].wait()
            _, _, out_hbm, c = specs[j]
            writes[j] = pltpu.async_copy(
                bufs[j % _NBUF],
                out_hbm.at[pl.ds(base + c * _CHUNK, _CHUNK)], wsems[j])
        for j in range(_NGATH - _NBUF, _NGATH):
            writes[j].wait()

    return gather2


_TILE = 4096

_DNUM = (((1,), (1,)), ((), ()))


def _mlp_body(u2_ref, m2_ref, w1_ref, b1_ref, w2_ref, b2_ref, out_ref):
    w1 = w1_ref[...]
    h = (lax.dot_general(u2_ref[...], w1[:, :D], _DNUM,
                         preferred_element_type=jnp.float32)
         + lax.dot_general(m2_ref[...], w1[:, D:], _DNUM,
                           preferred_element_type=jnp.float32)
         + b1_ref[...])
    h = jnp.maximum(h, 0.0)
    z = jnp.sum(h * w2_ref[...], axis=1) + b2_ref[0, 0]
    r = jax.nn.sigmoid(z) * (MAX_RATING - MIN_RATING) + MIN_RATING
    out_ref[...] = r.reshape(_TILE // 128, 128)


def _mlp(u2, m2, w1, b1, w2, b2):
    grid = (B // _TILE,)
    return pl.pallas_call(
        _mlp_body,
        grid=grid,
        in_specs=[
            pl.BlockSpec((_TILE, D), lambda i: (i, 0)),
            pl.BlockSpec((_TILE, D), lambda i: (i, 0)),
            pl.BlockSpec((NH, 2 * D), lambda i: (0, 0)),
            pl.BlockSpec((1, NH), lambda i: (0, 0)),
            pl.BlockSpec((1, NH), lambda i: (0, 0)),
            pl.BlockSpec((1, 1), lambda i: (0, 0)),
        ],
        out_specs=pl.BlockSpec((_TILE // 128, 128), lambda i: (i, 0)),
        out_shape=jax.ShapeDtypeStruct((B // 128, 128), jnp.float32),
    )(u2, m2, w1, b1, w2, b2)


def kernel(users, movies, u_weight, m_weight, lin1_w, lin1_b, lin2_w, lin2_b):
    u2, m2 = _make_gather()(
        u_weight, m_weight,
        users.reshape(B // _CHUNK, _CHUNK), movies.reshape(B // _CHUNK, _CHUNK))
    b1 = lin1_b.reshape(1, NH)
    w2 = lin2_w.reshape(1, NH)
    b2 = lin2_b.reshape(1, 1)
    return _mlp(u2, m2, lin1_w, b1, w2, b2).reshape(B, 1)
